# baseline (speedup 1.0000x reference)
import functools

import jax
import jax.numpy as jnp
from jax import lax
from jax.experimental import pallas as pl
from jax.experimental.pallas import tpu as pltpu
from jax.experimental.pallas import tpu_sc as plsc

B = 64
S = 1025
SP = 1040
D = 768
K = 8
NC = 2
NS = 16
L = 16


NSPLIT = 3
DSUB = D // NSPLIT


def _dist_body(*refs):
    xs = refs[:NSPLIT]
    out_ref = refs[NSPLIT]
    gs = [r[0, pl.ds(0, 1), :] for r in xs]
    gn = jnp.sqrt(sum(jnp.sum(gp * gp) for gp in gs))
    for c in range(8):
        num = jnp.zeros((128, 1), jnp.float32)
        l2 = jnp.zeros((128, 1), jnp.float32)
        for r, gp in zip(xs, gs):
            rows = r[0, pl.ds(c * 128, 128), :]
            num = num + jnp.sum(rows * gp, axis=1, keepdims=True)
            l2 = l2 + jnp.sum(rows * rows, axis=1, keepdims=True)
        dist = num / jnp.maximum(gn * jnp.sqrt(l2), 1e-8)
        if c == 0:
            rid = lax.broadcasted_iota(jnp.int32, (128, 1), 0)
            dist = jnp.where(rid == 0, -jnp.inf, dist)
        out_ref[0, pl.ds(c * 128, 128), :] = dist
    num = 0.0
    l2 = 0.0
    for r, gp in zip(xs, gs):
        row = r[0, pl.ds(1024, 1), :]
        num = num + jnp.sum(row * gp)
        l2 = l2 + jnp.sum(row * row)
    d_last = num / jnp.maximum(gn * jnp.sqrt(l2), 1e-8)
    tid = lax.broadcasted_iota(jnp.int32, (16, 1), 0)
    tail = jnp.where(tid == 0, d_last, -jnp.inf)
    out_ref[0, pl.ds(1024, 16), :] = tail


def _distances(x):
    return pl.pallas_call(
        _dist_body,
        grid=(B,),
        in_specs=[
            pl.BlockSpec((1, S, DSUB), lambda b, j=j: (b, 0, j))
            for j in range(NSPLIT)
        ],
        out_specs=pl.BlockSpec((1, SP, 1), lambda b: (b, 0, 0)),
        out_shape=jax.ShapeDtypeStruct((B, SP, 1), jnp.float32),
    )(*([x] * NSPLIT))


@functools.lru_cache(maxsize=None)
def _topk_gather_kernel():
    @functools.partial(
        pl.kernel,
        out_type=jax.ShapeDtypeStruct((B, K, D), jnp.float32),
        mesh=plsc.VectorSubcoreMesh(core_axis_name="c", subcore_axis_name="s"),
        scratch_types=[
            pltpu.VMEM((SP,), jnp.float32),
            pltpu.VMEM((L,), jnp.int32),
            pltpu.VMEM((L, D), jnp.float32),
            pltpu.SemaphoreType.DMA,
        ],
        compiler_params=pltpu.CompilerParams(needs_layout_passes=False),
    )
    def _topk_gather(dist_hbm, x2d_hbm, out_hbm, dist_v, idx_v, rows_v, sem):
        wid = lax.axis_index("s") * NC + lax.axis_index("c")
        iota = jnp.arange(L, dtype=jnp.int32)
        for i in range(B // (NC * NS)):
            b = wid * (B // (NC * NS)) + i
            pltpu.sync_copy(dist_hbm.at[b], dist_v)
            tv, ti = plsc.sort_key_val(dist_v[pl.ds(0, L)], iota,
                                       descending=True)
            for j in range(1, SP // L):
                sv, si = plsc.sort_key_val(dist_v[pl.ds(j * L, L)],
                                           iota + (j * L), descending=True)
                rv = lax.rev(sv, (0,))
                ri = lax.rev(si, (0,))
                m = tv >= rv
                hi = jnp.where(m, tv, rv)
                hx = jnp.where(m, ti, ri)
                tv, ti = plsc.sort_key_val(hi, hx, descending=True)
            idx_v[...] = ti + b * S
            pltpu.async_copy(x2d_hbm.at[idx_v], rows_v, sem).wait()
            pltpu.sync_copy(rows_v.at[pl.ds(0, K)], out_hbm.at[b])

    return _topk_gather


def kernel(x, images):
    del images
    dist = _distances(x).reshape(B, SP)
    x2d = x.reshape(B * S, D)
    return _topk_gather_kernel()(dist, x2d)


# speedup vs baseline: 1.4528x; 1.4528x over previous
import functools

import jax
import jax.numpy as jnp
from jax import lax
from jax.experimental import pallas as pl
from jax.experimental.pallas import tpu as pltpu
from jax.experimental.pallas import tpu_sc as plsc

B = 64
S = 1025
SP = 1040
D = 768
K = 8
NC = 2
NS = 16
L = 16


NSPLIT = 3
DSUB = D // NSPLIT


def _dist_body(*refs):
    xs = refs[:NSPLIT]
    out_ref = refs[NSPLIT]
    gs = [r[0, pl.ds(0, 1), :] for r in xs]
    gn = jnp.sqrt(sum(jnp.sum(gp * gp) for gp in gs))
    for c in range(8):
        num = jnp.zeros((128, 1), jnp.float32)
        l2 = jnp.zeros((128, 1), jnp.float32)
        for r, gp in zip(xs, gs):
            rows = r[0, pl.ds(c * 128, 128), :]
            num = num + jnp.sum(rows * gp, axis=1, keepdims=True)
            l2 = l2 + jnp.sum(rows * rows, axis=1, keepdims=True)
        dist = num / jnp.maximum(gn * jnp.sqrt(l2), 1e-8)
        if c == 0:
            rid = lax.broadcasted_iota(jnp.int32, (128, 1), 0)
            dist = jnp.where(rid == 0, -jnp.inf, dist)
        out_ref[0, pl.ds(c * 128, 128), :] = dist
    num = 0.0
    l2 = 0.0
    for r, gp in zip(xs, gs):
        row = r[0, pl.ds(1024, 1), :]
        num = num + jnp.sum(row * gp)
        l2 = l2 + jnp.sum(row * row)
    d_last = num / jnp.maximum(gn * jnp.sqrt(l2), 1e-8)
    tid = lax.broadcasted_iota(jnp.int32, (16, 1), 0)
    tail = jnp.where(tid == 0, d_last, -jnp.inf)
    out_ref[0, pl.ds(1024, 16), :] = tail


def _distances(x):
    return pl.pallas_call(
        _dist_body,
        grid=(B,),
        in_specs=[
            pl.BlockSpec((1, S, DSUB), lambda b, j=j: (b, 0, j))
            for j in range(NSPLIT)
        ],
        out_specs=pl.BlockSpec((1, SP, 1), lambda b: (b, 0, 0)),
        out_shape=jax.ShapeDtypeStruct((B, SP, 1), jnp.float32),
    )(*([x] * NSPLIT))


@functools.lru_cache(maxsize=None)
def _topk_kernel():
    @functools.partial(
        pl.kernel,
        out_type=jax.ShapeDtypeStruct((B, L), jnp.int32),
        mesh=plsc.VectorSubcoreMesh(core_axis_name="c", subcore_axis_name="s"),
        scratch_types=[
            pltpu.VMEM((SP,), jnp.float32),
            pltpu.VMEM((L,), jnp.int32),
        ],
        compiler_params=pltpu.CompilerParams(needs_layout_passes=False),
    )
    def _topk(dist_hbm, out_hbm, dist_v, idx_v):
        wid = lax.axis_index("s") * NC + lax.axis_index("c")
        iota = jnp.arange(L, dtype=jnp.int32)
        for i in range(B // (NC * NS)):
            b = wid * (B // (NC * NS)) + i
            pltpu.sync_copy(dist_hbm.at[b], dist_v)
            tv, ti = plsc.sort_key_val(dist_v[pl.ds(0, L)], iota,
                                       descending=True)
            for j in range(1, SP // L):
                sv, si = plsc.sort_key_val(dist_v[pl.ds(j * L, L)],
                                           iota + (j * L), descending=True)
                rv = lax.rev(sv, (0,))
                ri = lax.rev(si, (0,))
                m = tv >= rv
                hi = jnp.where(m, tv, rv)
                hx = jnp.where(m, ti, ri)
                tv, ti = plsc.sort_key_val(hi, hx, descending=True)
            idx_v[...] = ti
            pltpu.sync_copy(idx_v, out_hbm.at[b])

    return _topk


def _gather_body(idx_ref, *refs):
    out_ref = refs[K]
    b = pl.program_id(0)
    for j in range(K):
        off = idx_ref[b, j] % 8
        out_ref[0, pl.ds(j, 1), :] = refs[j][0, pl.ds(off, 1), :]


def _crop_gather(idx, x):
    grid_spec = pltpu.PrefetchScalarGridSpec(
        num_scalar_prefetch=1,
        grid=(B,),
        in_specs=[
            pl.BlockSpec((1, 8, D),
                         lambda b, idx_ref, j=j: (b, idx_ref[b, j] // 8, 0))
            for j in range(K)
        ],
        out_specs=pl.BlockSpec((1, K, D), lambda b, idx_ref: (b, 0, 0)),
    )
    return pl.pallas_call(
        _gather_body,
        grid_spec=grid_spec,
        out_shape=jax.ShapeDtypeStruct((B, K, D), jnp.float32),
    )(idx, *([x] * K))


def kernel(x, images):
    del images
    dist = _distances(x).reshape(B, SP)
    idx = _topk_kernel()(dist)
    return _crop_gather(idx, x)


# speedup vs baseline: 3.7085x; 2.5527x over previous
import functools

import jax
import jax.numpy as jnp
from jax import lax
from jax.experimental import pallas as pl
from jax.experimental.pallas import tpu as pltpu
from jax.experimental.pallas import tpu_sc as plsc

B = 64
S = 1025
SP = 1152
D = 768
K = 8
NC = 2
NS = 16
L = 16
TCH = 128
BCH = 8


def _dist_body(y_ref, g_ref, out_ref):
    c = pl.program_id(0)
    g = g_ref[...]
    gn = jnp.sqrt(jnp.sum(g * g, axis=2))
    rows = y_ref[...]
    num = jnp.sum(rows * g, axis=2)
    ln = jnp.sqrt(jnp.sum(rows * rows, axis=2))
    dist = num / jnp.maximum(gn * ln, 1e-8)
    rid = c * TCH + lax.broadcasted_iota(jnp.int32, (TCH, BCH), 0)
    valid = (rid > 0) & (rid < S)
    dist = jnp.where(valid, dist, -jnp.inf)
    out_ref[...] = jnp.swapaxes(dist, 0, 1)


def _distances(y):
    return pl.pallas_call(
        _dist_body,
        grid=(SP // TCH, B // BCH),
        in_specs=[
            pl.BlockSpec((TCH, BCH, D), lambda c, bc: (c, bc, 0)),
            pl.BlockSpec((1, BCH, D), lambda c, bc: (0, bc, 0)),
        ],
        out_specs=pl.BlockSpec((BCH, TCH), lambda c, bc: (bc, c)),
        out_shape=jax.ShapeDtypeStruct((B, SP), jnp.float32),
    )(y, y)


@functools.lru_cache(maxsize=None)
def _topk_crop_kernel():
    @functools.partial(
        pl.kernel,
        out_type=jax.ShapeDtypeStruct((B, K, D), jnp.float32),
        mesh=plsc.VectorSubcoreMesh(core_axis_name="c", subcore_axis_name="s"),
        scratch_types=[
            pltpu.VMEM((SP,), jnp.float32),
            pltpu.VMEM((L,), jnp.int32),
            pltpu.VMEM((L, D), jnp.float32),
            pltpu.SemaphoreType.DMA,
        ],
        compiler_params=pltpu.CompilerParams(needs_layout_passes=False),
    )
    def _topk_crop(dist_hbm, y2d_hbm, out_hbm, dist_v, idx_v, rows_v, sem):
        wid = lax.axis_index("s") * NC + lax.axis_index("c")
        iota = jnp.arange(L, dtype=jnp.int32)
        for i in range(B // (NC * NS)):
            b = wid * (B // (NC * NS)) + i
            pltpu.sync_copy(dist_hbm.at[b], dist_v)
            tv, ti = plsc.sort_key_val(dist_v[pl.ds(0, L)], iota,
                                       descending=True)
            for j in range(1, SP // L):
                sv, si = plsc.sort_key_val(dist_v[pl.ds(j * L, L)],
                                           iota + (j * L), descending=True)
                rv = lax.rev(sv, (0,))
                ri = lax.rev(si, (0,))
                m = tv >= rv
                hi = jnp.where(m, tv, rv)
                hx = jnp.where(m, ti, ri)
                tv, ti = plsc.sort_key_val(hi, hx, descending=True)
            idx_v[...] = ti * B + b
            pltpu.async_copy(y2d_hbm.at[idx_v], rows_v, sem).wait()
            pltpu.sync_copy(rows_v.at[pl.ds(0, K)], out_hbm.at[b])

    return _topk_crop


def kernel(x, images):
    del images
    y = jnp.transpose(x, (1, 0, 2))
    dist = _distances(y)
    y2d = y.reshape(S * B, D)
    return _topk_crop_kernel()(dist, y2d)


# speedup vs baseline: 3.7608x; 1.0141x over previous
import functools

import jax
import jax.numpy as jnp
from jax import lax
from jax.experimental import pallas as pl
from jax.experimental.pallas import tpu as pltpu
from jax.experimental.pallas import tpu_sc as plsc

B = 64
S = 1025
SP = 1152
D = 768
K = 8
NC = 2
NS = 16
L = 16
TCH = 128
BCH = 8


HCH = TCH // 2


def _dist_body(ya_ref, yb_ref, g_ref, out_ref):
    c = pl.program_id(0)
    g = g_ref[...]
    gn = jnp.sqrt(jnp.sum(g * g, axis=2))
    halves = []
    for h, y_ref in enumerate((ya_ref, yb_ref)):
        rows = y_ref[...]
        num = jnp.sum(rows * g, axis=2)
        ln = jnp.sqrt(jnp.sum(rows * rows, axis=2))
        dist = num / jnp.maximum(gn * ln, 1e-8)
        rid = (c * TCH + h * HCH
               + lax.broadcasted_iota(jnp.int32, (HCH, BCH), 0))
        valid = (rid > 0) & (rid < S)
        halves.append(jnp.where(valid, dist, -jnp.inf))
    dist = jnp.concatenate(halves, axis=0)
    out_ref[...] = jnp.swapaxes(dist, 0, 1)


def _distances(y):
    return pl.pallas_call(
        _dist_body,
        grid=(SP // TCH, B // BCH),
        in_specs=[
            pl.BlockSpec((HCH, BCH, D), lambda c, bc: (2 * c, bc, 0)),
            pl.BlockSpec((HCH, BCH, D),
                         lambda c, bc: (jnp.minimum(2 * c + 1, 2 * (SP // TCH) - 2),
                                        bc, 0)),
            pl.BlockSpec((1, BCH, D), lambda c, bc: (0, bc, 0)),
        ],
        out_specs=pl.BlockSpec((BCH, TCH), lambda c, bc: (bc, c)),
        out_shape=jax.ShapeDtypeStruct((B, SP), jnp.float32),
    )(y, y, y)


@functools.lru_cache(maxsize=None)
def _topk_crop_kernel():
    @functools.partial(
        pl.kernel,
        out_type=jax.ShapeDtypeStruct((B, K, D), jnp.float32),
        mesh=plsc.VectorSubcoreMesh(core_axis_name="c", subcore_axis_name="s"),
        scratch_types=[
            pltpu.VMEM((SP,), jnp.float32),
            pltpu.VMEM((L,), jnp.int32),
            pltpu.VMEM((L, D), jnp.float32),
            pltpu.SemaphoreType.DMA,
        ],
        compiler_params=pltpu.CompilerParams(needs_layout_passes=False),
    )
    def _topk_crop(dist_hbm, y2d_hbm, out_hbm, dist_v, idx_v, rows_v, sem):
        wid = lax.axis_index("s") * NC + lax.axis_index("c")
        iota = jnp.arange(L, dtype=jnp.int32)
        for i in range(B // (NC * NS)):
            b = wid * (B // (NC * NS)) + i
            pltpu.sync_copy(dist_hbm.at[b], dist_v)
            tv, ti = plsc.sort_key_val(dist_v[pl.ds(0, L)], iota,
                                       descending=True)
            for j in range(1, SP // L):
                sv, si = plsc.sort_key_val(dist_v[pl.ds(j * L, L)],
                                           iota + (j * L), descending=True)
                rv = lax.rev(sv, (0,))
                ri = lax.rev(si, (0,))
                m = tv >= rv
                hi = jnp.where(m, tv, rv)
                hx = jnp.where(m, ti, ri)
                tv, ti = plsc.sort_key_val(hi, hx, descending=True)
            idx_v[...] = ti * B + b
            pltpu.async_copy(y2d_hbm.at[idx_v], rows_v, sem).wait()
            pltpu.sync_copy(rows_v.at[pl.ds(0, K)], out_hbm.at[b])

    return _topk_crop


def kernel(x, images):
    del images
    y = jnp.transpose(x, (1, 0, 2))
    dist = _distances(y)
    y2d = y.reshape(S * B, D)
    return _topk_crop_kernel()(dist, y2d)


# speedup vs baseline: 4.7727x; 1.2691x over previous
import functools

import jax
import jax.numpy as jnp
from jax import lax
from jax.experimental import pallas as pl
from jax.experimental.pallas import tpu as pltpu
from jax.experimental.pallas import tpu_sc as plsc

B = 64
S = 1025
SP = 1152
D = 768
K = 8
NC = 2
NS = 16
L = 16
TCH = 128
BCH = 32


HCH = TCH // 2


def _dist_body(ya_ref, yb_ref, g_ref, out_ref):
    c = pl.program_id(0)
    g = g_ref[...]
    gn = jnp.sqrt(jnp.sum(g * g, axis=2))
    halves = []
    for h, y_ref in enumerate((ya_ref, yb_ref)):
        rows = y_ref[...]
        num = jnp.sum(rows * g, axis=2)
        ln = jnp.sqrt(jnp.sum(rows * rows, axis=2))
        dist = num / jnp.maximum(gn * ln, 1e-8)
        rid = (c * TCH + h * HCH
               + lax.broadcasted_iota(jnp.int32, (HCH, BCH), 0))
        valid = (rid > 0) & (rid < S)
        halves.append(jnp.where(valid, dist, -jnp.inf))
    dist = jnp.concatenate(halves, axis=0)
    out_ref[...] = jnp.swapaxes(dist, 0, 1)


def _distances(y):
    return pl.pallas_call(
        _dist_body,
        grid=(SP // TCH, B // BCH),
        in_specs=[
            pl.BlockSpec((HCH, BCH, D), lambda c, bc: (2 * c, bc, 0)),
            pl.BlockSpec((HCH, BCH, D),
                         lambda c, bc: (jnp.minimum(2 * c + 1, 2 * (SP // TCH) - 2),
                                        bc, 0)),
            pl.BlockSpec((1, BCH, D), lambda c, bc: (0, bc, 0)),
        ],
        out_specs=pl.BlockSpec((BCH, TCH), lambda c, bc: (bc, c)),
        out_shape=jax.ShapeDtypeStruct((B, SP), jnp.float32),
    )(y, y, y)


@functools.lru_cache(maxsize=None)
def _topk_crop_kernel():
    @functools.partial(
        pl.kernel,
        out_type=jax.ShapeDtypeStruct((B, K, D), jnp.float32),
        mesh=plsc.VectorSubcoreMesh(core_axis_name="c", subcore_axis_name="s"),
        scratch_types=[
            pltpu.VMEM((SP,), jnp.float32),
            pltpu.VMEM((L,), jnp.int32),
            pltpu.VMEM((L, D), jnp.float32),
            pltpu.SemaphoreType.DMA,
        ],
        compiler_params=pltpu.CompilerParams(needs_layout_passes=False),
    )
    def _topk_crop(dist_hbm, y2d_hbm, out_hbm, dist_v, idx_v, rows_v, sem):
        wid = lax.axis_index("s") * NC + lax.axis_index("c")
        iota = jnp.arange(L, dtype=jnp.int32)
        for i in range(B // (NC * NS)):
            b = wid * (B // (NC * NS)) + i
            pltpu.sync_copy(dist_hbm.at[b], dist_v)
            tv, ti = plsc.sort_key_val(dist_v[pl.ds(0, L)], iota,
                                       descending=True)
            for j in range(1, SP // L):
                sv, si = plsc.sort_key_val(dist_v[pl.ds(j * L, L)],
                                           iota + (j * L), descending=True)
                rv = lax.rev(sv, (0,))
                ri = lax.rev(si, (0,))
                m = tv >= rv
                hi = jnp.where(m, tv, rv)
                hx = jnp.where(m, ti, ri)
                tv, ti = plsc.sort_key_val(hi, hx, descending=True)
            idx_v[...] = ti * B + b
            pltpu.async_copy(y2d_hbm.at[idx_v], rows_v, sem).wait()
            pltpu.sync_copy(rows_v.at[pl.ds(0, K)], out_hbm.at[b])

    return _topk_crop


def kernel(x, images):
    del images
    y = jnp.transpose(x, (1, 0, 2))
    dist = _distances(y)
    y2d = y.reshape(S * B, D)
    return _topk_crop_kernel()(dist, y2d)
